# baseline (speedup 1.0000x reference)
import functools
from itertools import product

import jax
import jax.numpy as jnp
import numpy as np
from jax.experimental import pallas as pl
from jax.experimental.pallas import tpu as pltpu

_B = 4096
_C = 6
_D = 512
_DOM_LEN = 64
_TMP_LEN = 720
_BS = 128
_NBUF = 4
_LOOK = 3


def _in_copies(s_hbm, t_hbm, s_buf, t_buf, insem, slot, step):
    cs = pltpu.make_async_copy(
        s_hbm.at[pl.ds(step * _BS, _BS)], s_buf.at[slot], insem.at[slot, 0])
    ct = pltpu.make_async_copy(
        t_hbm.at[pl.ds(step * _BS, _BS)], t_buf.at[slot], insem.at[slot, 1])
    return cs, ct


def _out_copies(s_buf, t_buf, out_hbm, outsem, slot, step):
    c1 = pltpu.make_async_copy(
        s_buf.at[slot], out_hbm.at[pl.ds(step * _BS, _BS)], outsem.at[slot, 0])
    c2 = pltpu.make_async_copy(
        t_buf.at[slot], out_hbm.at[pl.ds(_B + step * _BS, _BS)],
        outsem.at[slot, 1])
    return c1, c2


def _select_kernel(mask_ref, s_hbm, t_hbm, out_hbm,
                   s_buf, t_buf, insem, outsem):
    b = pl.program_id(0)
    nb = pl.num_programs(0)
    slot = jax.lax.rem(b, _NBUF)

    @pl.when(b == 0)
    def _():
        for k in range(_LOOK):
            cs, ct = _in_copies(s_hbm, t_hbm, s_buf, t_buf, insem, k, k)
            cs.start()
            ct.start()

    tgt = b + _LOOK

    @pl.when(tgt < nb)
    def _():
        tslot = jax.lax.rem(tgt, _NBUF)

        @pl.when(tgt >= _NBUF)
        def _():
            c1, c2 = _out_copies(s_buf, t_buf, out_hbm, outsem, tslot,
                                 tgt - _NBUF)
            c1.wait()
            c2.wait()

        cs, ct = _in_copies(s_hbm, t_hbm, s_buf, t_buf, insem, tslot, tgt)
        cs.start()
        ct.start()

    cs, ct = _in_copies(s_hbm, t_hbm, s_buf, t_buf, insem, slot, b)
    cs.wait()
    ct.wait()

    m = mask_ref[...]
    s = s_buf[slot]
    t = t_buf[slot]
    d = m * (t - s)
    s_buf[slot] = s + d
    t_buf[slot] = t - d

    c1, c2 = _out_copies(s_buf, t_buf, out_hbm, outsem, slot, b)
    c1.start()
    c2.start()

    @pl.when(b == nb - 1)
    def _():
        for k in range(_NBUF):
            step = nb - _NBUF + k
            c1, c2 = _out_copies(s_buf, t_buf, out_hbm, outsem,
                                 step % _NBUF, step)
            c1.wait()
            c2.wait()


@functools.partial(jax.jit, static_argnums=())
def _labels():
    rkey = jax.random.key(1234)
    ka, kb = jax.random.split(rkey)
    tem_rand_lab = jax.random.randint(ka, (_B,), 0, _TMP_LEN)
    dom_rand_lab1 = jax.random.randint(kb, (_B,), 0, _DOM_LEN // 2)
    return tem_rand_lab, dom_rand_lab1


def kernel(sfeat, tfeat):
    B, C, D = _B, _C, _D
    tem_rand_lab, dom_rand_lab1 = _labels()
    dom_set = jnp.asarray(
        np.array(list(product(*[[0, 1]] * C)), dtype=np.int32))
    bits = jnp.take(dom_set, dom_rand_lab1, axis=0)
    mask = bits.astype(jnp.float32)[:, :, None]

    nb = B // _BS
    feat = pl.pallas_call(
        _select_kernel,
        grid=(nb,),
        in_specs=[
            pl.BlockSpec((_BS, C, 1), lambda b: (b, 0, 0)),
            pl.BlockSpec(memory_space=pltpu.MemorySpace.HBM),
            pl.BlockSpec(memory_space=pltpu.MemorySpace.HBM),
        ],
        out_specs=pl.BlockSpec(memory_space=pltpu.MemorySpace.HBM),
        out_shape=jax.ShapeDtypeStruct((2 * B, C, D), sfeat.dtype),
        scratch_shapes=[
            pltpu.VMEM((_NBUF, _BS, C, D), jnp.float32),
            pltpu.VMEM((_NBUF, _BS, C, D), jnp.float32),
            pltpu.SemaphoreType.DMA((_NBUF, 2)),
            pltpu.SemaphoreType.DMA((_NBUF, 2)),
        ],
        compiler_params=pltpu.CompilerParams(
            dimension_semantics=("arbitrary",),
            needs_layout_passes=True),
    )(mask, sfeat, tfeat)

    dom_lab = jnp.concatenate([dom_rand_lab1, _DOM_LEN - 1 - dom_rand_lab1])
    tmp_lab = jnp.concatenate([tem_rand_lab, tem_rand_lab])
    dom_conf_lab = jnp.full((2 * B, _DOM_LEN), 1.0 / _DOM_LEN, jnp.float32)
    tmp_conf_lab = jnp.full((2 * B, _TMP_LEN), 1.0 / _TMP_LEN, jnp.float32)
    return (feat, dom_lab, dom_conf_lab, tmp_lab, tmp_conf_lab)


# speedup vs baseline: 2.0866x; 2.0866x over previous
import functools
from itertools import product

import jax
import jax.numpy as jnp
import numpy as np
from jax.experimental import pallas as pl
from jax.experimental.pallas import tpu as pltpu

_B = 4096
_C = 6
_D = 512
_DOM_LEN = 64
_TMP_LEN = 720
_BS = 512
_NB = _B // _BS
_NBUF = 4
_LOOK = 3


def _split(step):
    return jax.lax.div(step, _NB), jax.lax.rem(step, _NB)


def _in_copies(s_hbm, t_hbm, s_buf, t_buf, insem, slot, step):
    j, b = _split(step)
    cs = pltpu.make_async_copy(
        s_hbm.at[j, pl.ds(b * _BS, _BS)], s_buf.at[slot], insem.at[slot, 0])
    ct = pltpu.make_async_copy(
        t_hbm.at[j, pl.ds(b * _BS, _BS)], t_buf.at[slot], insem.at[slot, 1])
    return cs, ct


def _out_copies(s_buf, t_buf, out_hbm, outsem, slot, step):
    j, b = _split(step)
    c1 = pltpu.make_async_copy(
        s_buf.at[slot], out_hbm.at[j, 0, pl.ds(b * _BS, _BS)],
        outsem.at[slot, 0])
    c2 = pltpu.make_async_copy(
        t_buf.at[slot], out_hbm.at[j, 1, pl.ds(b * _BS, _BS)],
        outsem.at[slot, 1])
    return c1, c2


def _select_kernel(mask_ref, s_hbm, t_hbm, out_hbm,
                   s_buf, t_buf, insem, outsem):
    i = pl.program_id(0)
    n = pl.num_programs(0)
    slot = jax.lax.rem(i, _NBUF)

    @pl.when(i == 0)
    def _():
        for k in range(_LOOK):
            cs, ct = _in_copies(s_hbm, t_hbm, s_buf, t_buf, insem, k, k)
            cs.start()
            ct.start()

    tgt = i + _LOOK

    @pl.when(tgt < n)
    def _():
        tslot = jax.lax.rem(tgt, _NBUF)

        @pl.when(tgt >= _NBUF)
        def _():
            c1, c2 = _out_copies(s_buf, t_buf, out_hbm, outsem, tslot,
                                 tgt - _NBUF)
            c1.wait()
            c2.wait()

        cs, ct = _in_copies(s_hbm, t_hbm, s_buf, t_buf, insem, tslot, tgt)
        cs.start()
        ct.start()

    cs, ct = _in_copies(s_hbm, t_hbm, s_buf, t_buf, insem, slot, i)
    cs.wait()
    ct.wait()

    m = mask_ref[0]
    s = s_buf[slot]
    t = t_buf[slot]
    d = m * (t - s)
    s_buf[slot] = s + d
    t_buf[slot] = t - d

    c1, c2 = _out_copies(s_buf, t_buf, out_hbm, outsem, slot, i)
    c1.start()
    c2.start()

    @pl.when(i == n - 1)
    def _():
        for k in range(_NBUF):
            step = _C * _NB - _NBUF + k
            c1, c2 = _out_copies(s_buf, t_buf, out_hbm, outsem,
                                 step % _NBUF, step)
            c1.wait()
            c2.wait()


@functools.partial(jax.jit, static_argnums=())
def _labels():
    rkey = jax.random.key(1234)
    ka, kb = jax.random.split(rkey)
    tem_rand_lab = jax.random.randint(ka, (_B,), 0, _TMP_LEN)
    dom_rand_lab1 = jax.random.randint(kb, (_B,), 0, _DOM_LEN // 2)
    return tem_rand_lab, dom_rand_lab1


def kernel(sfeat, tfeat):
    B, C, D = _B, _C, _D
    tem_rand_lab, dom_rand_lab1 = _labels()
    dom_set = jnp.asarray(
        np.array(list(product(*[[0, 1]] * C)), dtype=np.int32))
    bits = jnp.take(dom_set, dom_rand_lab1, axis=0)
    mask = bits.T.astype(jnp.float32)[:, :, None]

    sT = jnp.transpose(sfeat, (1, 0, 2))
    tT = jnp.transpose(tfeat, (1, 0, 2))

    n = C * _NB
    outT = pl.pallas_call(
        _select_kernel,
        grid=(n,),
        in_specs=[
            pl.BlockSpec((1, _BS, 1), lambda i: (i // _NB, i % _NB, 0)),
            pl.BlockSpec(memory_space=pltpu.MemorySpace.HBM),
            pl.BlockSpec(memory_space=pltpu.MemorySpace.HBM),
        ],
        out_specs=pl.BlockSpec(memory_space=pltpu.MemorySpace.HBM),
        out_shape=jax.ShapeDtypeStruct((C, 2, B, D), sfeat.dtype),
        scratch_shapes=[
            pltpu.VMEM((_NBUF, _BS, _D), jnp.float32),
            pltpu.VMEM((_NBUF, _BS, _D), jnp.float32),
            pltpu.SemaphoreType.DMA((_NBUF, 2)),
            pltpu.SemaphoreType.DMA((_NBUF, 2)),
        ],
        compiler_params=pltpu.CompilerParams(
            dimension_semantics=("arbitrary",)),
    )(mask, sT, tT)
    feat = jnp.transpose(outT.reshape(C, 2 * B, D), (1, 0, 2))

    dom_lab = jnp.concatenate([dom_rand_lab1, _DOM_LEN - 1 - dom_rand_lab1])
    tmp_lab = jnp.concatenate([tem_rand_lab, tem_rand_lab])
    dom_conf_lab = jnp.full((2 * B, _DOM_LEN), 1.0 / _DOM_LEN, jnp.float32)
    tmp_conf_lab = jnp.full((2 * B, _TMP_LEN), 1.0 / _TMP_LEN, jnp.float32)
    return (feat, dom_lab, dom_conf_lab, tmp_lab, tmp_conf_lab)


# speedup vs baseline: 3.4937x; 1.6743x over previous
import functools
from itertools import product

import jax
import jax.numpy as jnp
import numpy as np
from jax.experimental import pallas as pl
from jax.experimental.pallas import tpu as pltpu

_B = 4096
_C = 6
_D = 512
_DOM_LEN = 64
_TMP_LEN = 720
_BS = 512
_NB = _B // _BS
_NBUF = 4
_LOOK = 3


def _split(step):
    return jax.lax.div(step, _NB), jax.lax.rem(step, _NB)


def _in_copies(s_hbm, t_hbm, s_buf, t_buf, insem, slot, step):
    j, b = _split(step)
    cs = pltpu.make_async_copy(
        s_hbm.at[j, pl.ds(b * _BS, _BS)], s_buf.at[slot], insem.at[slot, 0])
    ct = pltpu.make_async_copy(
        t_hbm.at[j, pl.ds(b * _BS, _BS)], t_buf.at[slot], insem.at[slot, 1])
    return cs, ct


def _out_copies(s_buf, t_buf, out_hbm, outsem, slot, step):
    j, b = _split(step)
    c1 = pltpu.make_async_copy(
        s_buf.at[slot], out_hbm.at[j, 0, pl.ds(b * _BS, _BS)],
        outsem.at[slot, 0])
    c2 = pltpu.make_async_copy(
        t_buf.at[slot], out_hbm.at[j, 1, pl.ds(b * _BS, _BS)],
        outsem.at[slot, 1])
    return c1, c2


def _select_kernel(mask_ref, s_hbm, t_hbm, out_hbm,
                   s_buf, t_buf, insem, outsem):
    i = pl.program_id(0)
    n = pl.num_programs(0)
    slot = jax.lax.rem(i, _NBUF)

    @pl.when(i == 0)
    def _():
        for k in range(_LOOK):
            cs, ct = _in_copies(s_hbm, t_hbm, s_buf, t_buf, insem, k, k)
            cs.start()
            ct.start()

    tgt = i + _LOOK

    @pl.when(tgt < n)
    def _():
        tslot = jax.lax.rem(tgt, _NBUF)

        @pl.when(tgt >= _NBUF)
        def _():
            c1, c2 = _out_copies(s_buf, t_buf, out_hbm, outsem, tslot,
                                 tgt - _NBUF)
            c1.wait()
            c2.wait()

        cs, ct = _in_copies(s_hbm, t_hbm, s_buf, t_buf, insem, tslot, tgt)
        cs.start()
        ct.start()

    cs, ct = _in_copies(s_hbm, t_hbm, s_buf, t_buf, insem, slot, i)
    cs.wait()
    ct.wait()

    m = mask_ref[0][:, 0:1]
    s = s_buf[slot]
    t = t_buf[slot]
    d = m * (t - s)
    s_buf[slot] = s + d
    t_buf[slot] = t - d

    c1, c2 = _out_copies(s_buf, t_buf, out_hbm, outsem, slot, i)
    c1.start()
    c2.start()

    @pl.when(i == n - 1)
    def _():
        for k in range(_NBUF):
            step = _C * _NB - _NBUF + k
            c1, c2 = _out_copies(s_buf, t_buf, out_hbm, outsem,
                                 step % _NBUF, step)
            c1.wait()
            c2.wait()


def _labels():
    rkey = jax.random.key(1234)
    ka, kb = jax.random.split(rkey)
    tem_rand_lab = jax.random.randint(ka, (_B,), 0, _TMP_LEN)
    dom_rand_lab1 = jax.random.randint(kb, (_B,), 0, _DOM_LEN // 2)
    return tem_rand_lab, dom_rand_lab1


with jax.default_device(jax.devices("cpu")[0]):
    _TEM_LAB, _DOM_LAB1 = (np.asarray(x) for x in jax.jit(_labels)())

_DOM_SET = np.array(list(product(*[[0, 1]] * _C)), dtype=np.int32)
_BITS = _DOM_SET[_DOM_LAB1]
_MASK = np.ascontiguousarray(
    np.broadcast_to(_BITS.T.astype(np.float32)[:, :, None], (_C, _B, 128)))
_DOM_LAB = np.concatenate([_DOM_LAB1, _DOM_LEN - 1 - _DOM_LAB1]).astype(np.int32)
_TMP_LAB = np.concatenate([_TEM_LAB, _TEM_LAB]).astype(np.int32)


def kernel(sfeat, tfeat):
    B, C, D = _B, _C, _D
    mask = jnp.asarray(_MASK)

    sT = jnp.transpose(sfeat, (1, 0, 2))
    tT = jnp.transpose(tfeat, (1, 0, 2))

    n = C * _NB
    outT = pl.pallas_call(
        _select_kernel,
        grid=(n,),
        in_specs=[
            pl.BlockSpec((1, _BS, 128), lambda i: (i // _NB, i % _NB, 0)),
            pl.BlockSpec(memory_space=pltpu.MemorySpace.HBM),
            pl.BlockSpec(memory_space=pltpu.MemorySpace.HBM),
        ],
        out_specs=pl.BlockSpec(memory_space=pltpu.MemorySpace.HBM),
        out_shape=jax.ShapeDtypeStruct((C, 2, B, D), sfeat.dtype),
        scratch_shapes=[
            pltpu.VMEM((_NBUF, _BS, _D), jnp.float32),
            pltpu.VMEM((_NBUF, _BS, _D), jnp.float32),
            pltpu.SemaphoreType.DMA((_NBUF, 2)),
            pltpu.SemaphoreType.DMA((_NBUF, 2)),
        ],
        compiler_params=pltpu.CompilerParams(
            dimension_semantics=("arbitrary",)),
    )(mask, sT, tT)
    feat = jnp.transpose(outT.reshape(C, 2 * B, D), (1, 0, 2))

    dom_lab = jnp.asarray(_DOM_LAB)
    tmp_lab = jnp.asarray(_TMP_LAB)
    dom_conf_lab = jnp.full((2 * B, _DOM_LEN), 1.0 / _DOM_LEN, jnp.float32)
    tmp_conf_lab = jnp.full((2 * B, _TMP_LEN), 1.0 / _TMP_LEN, jnp.float32)
    return (feat, dom_lab, dom_conf_lab, tmp_lab, tmp_conf_lab)


# speedup vs baseline: 4.2961x; 1.2297x over previous
import functools
from itertools import product

import jax
import jax.numpy as jnp
import numpy as np
from jax.experimental import pallas as pl
from jax.experimental.pallas import tpu as pltpu

_B = 4096
_C = 6
_D = 512
_DOM_LEN = 64
_TMP_LEN = 720
_BS = 1024
_NB = _B // _BS
_NBUF = 4
_LOOK = 3


def _split(step):
    return jax.lax.div(step, _NB), jax.lax.rem(step, _NB)


def _in_copies(s_hbm, t_hbm, s_buf, t_buf, insem, slot, step):
    j, b = _split(step)
    cs = pltpu.make_async_copy(
        s_hbm.at[j, pl.ds(b * _BS, _BS)], s_buf.at[slot], insem.at[slot, 0])
    ct = pltpu.make_async_copy(
        t_hbm.at[j, pl.ds(b * _BS, _BS)], t_buf.at[slot], insem.at[slot, 1])
    return cs, ct


def _out_copies(s_buf, t_buf, out_hbm, outsem, slot, step):
    j, b = _split(step)
    c1 = pltpu.make_async_copy(
        s_buf.at[slot], out_hbm.at[j, 0, pl.ds(b * _BS, _BS)],
        outsem.at[slot, 0])
    c2 = pltpu.make_async_copy(
        t_buf.at[slot], out_hbm.at[j, 1, pl.ds(b * _BS, _BS)],
        outsem.at[slot, 1])
    return c1, c2


def _select_kernel(mask_ref, s_hbm, t_hbm, out_hbm,
                   s_buf, t_buf, insem, outsem):
    i = pl.program_id(0)
    n = pl.num_programs(0)
    slot = jax.lax.rem(i, _NBUF)

    @pl.when(i == 0)
    def _():
        for k in range(_LOOK):
            cs, ct = _in_copies(s_hbm, t_hbm, s_buf, t_buf, insem, k, k)
            cs.start()
            ct.start()

    tgt = i + _LOOK

    @pl.when(tgt < n)
    def _():
        tslot = jax.lax.rem(tgt, _NBUF)

        @pl.when(tgt >= _NBUF)
        def _():
            c1, c2 = _out_copies(s_buf, t_buf, out_hbm, outsem, tslot,
                                 tgt - _NBUF)
            c1.wait()
            c2.wait()

        cs, ct = _in_copies(s_hbm, t_hbm, s_buf, t_buf, insem, tslot, tgt)
        cs.start()
        ct.start()

    cs, ct = _in_copies(s_hbm, t_hbm, s_buf, t_buf, insem, slot, i)
    cs.wait()
    ct.wait()

    m = mask_ref[0][:, 0:1]
    s = s_buf[slot]
    t = t_buf[slot]
    d = m * (t - s)
    s_buf[slot] = s + d
    t_buf[slot] = t - d

    c1, c2 = _out_copies(s_buf, t_buf, out_hbm, outsem, slot, i)
    c1.start()
    c2.start()

    @pl.when(i == n - 1)
    def _():
        for k in range(_NBUF):
            step = _C * _NB - _NBUF + k
            c1, c2 = _out_copies(s_buf, t_buf, out_hbm, outsem,
                                 step % _NBUF, step)
            c1.wait()
            c2.wait()


def _labels():
    rkey = jax.random.key(1234)
    ka, kb = jax.random.split(rkey)
    tem_rand_lab = jax.random.randint(ka, (_B,), 0, _TMP_LEN)
    dom_rand_lab1 = jax.random.randint(kb, (_B,), 0, _DOM_LEN // 2)
    return tem_rand_lab, dom_rand_lab1


with jax.default_device(jax.devices("cpu")[0]):
    _TEM_LAB, _DOM_LAB1 = (np.asarray(x) for x in jax.jit(_labels)())

_DOM_SET = np.array(list(product(*[[0, 1]] * _C)), dtype=np.int32)
_BITS = _DOM_SET[_DOM_LAB1]
_MASK = np.ascontiguousarray(
    np.broadcast_to(_BITS.T.astype(np.float32)[:, :, None], (_C, _B, 128)))
_DOM_LAB = np.concatenate([_DOM_LAB1, _DOM_LEN - 1 - _DOM_LAB1]).astype(np.int32)
_TMP_LAB = np.concatenate([_TEM_LAB, _TEM_LAB]).astype(np.int32)


def kernel(sfeat, tfeat):
    B, C, D = _B, _C, _D
    mask = jnp.asarray(_MASK)

    sT = jnp.transpose(sfeat, (1, 0, 2))
    tT = jnp.transpose(tfeat, (1, 0, 2))

    n = C * _NB
    outT = pl.pallas_call(
        _select_kernel,
        grid=(n,),
        in_specs=[
            pl.BlockSpec((1, _BS, 128), lambda i: (i // _NB, i % _NB, 0)),
            pl.BlockSpec(memory_space=pltpu.MemorySpace.HBM),
            pl.BlockSpec(memory_space=pltpu.MemorySpace.HBM),
        ],
        out_specs=pl.BlockSpec(memory_space=pltpu.MemorySpace.HBM),
        out_shape=jax.ShapeDtypeStruct((C, 2, B, D), sfeat.dtype),
        scratch_shapes=[
            pltpu.VMEM((_NBUF, _BS, _D), jnp.float32),
            pltpu.VMEM((_NBUF, _BS, _D), jnp.float32),
            pltpu.SemaphoreType.DMA((_NBUF, 2)),
            pltpu.SemaphoreType.DMA((_NBUF, 2)),
        ],
        compiler_params=pltpu.CompilerParams(
            dimension_semantics=("arbitrary",)),
    )(mask, sT, tT)
    feat = jnp.transpose(outT.reshape(C, 2 * B, D), (1, 0, 2))

    dom_lab = jnp.asarray(_DOM_LAB)
    tmp_lab = jnp.asarray(_TMP_LAB)
    dom_conf_lab = jnp.full((2 * B, _DOM_LEN), 1.0 / _DOM_LEN, jnp.float32)
    tmp_conf_lab = jnp.full((2 * B, _TMP_LEN), 1.0 / _TMP_LEN, jnp.float32)
    return (feat, dom_lab, dom_conf_lab, tmp_lab, tmp_conf_lab)


# speedup vs baseline: 4.5035x; 1.0483x over previous
import functools
from itertools import product

import jax
import jax.numpy as jnp
import numpy as np
from jax.experimental import pallas as pl
from jax.experimental.pallas import tpu as pltpu

_B = 4096
_C = 6
_D = 512
_DOM_LEN = 64
_TMP_LEN = 720
_BS = 2048
_NB = _B // _BS
_NBUF = 4
_LOOK = 3


def _split(step):
    return jax.lax.div(step, _NB), jax.lax.rem(step, _NB)


def _in_copies(s_hbm, t_hbm, s_buf, t_buf, insem, slot, step):
    j, b = _split(step)
    cs = pltpu.make_async_copy(
        s_hbm.at[j, pl.ds(b * _BS, _BS)], s_buf.at[slot], insem.at[slot, 0])
    ct = pltpu.make_async_copy(
        t_hbm.at[j, pl.ds(b * _BS, _BS)], t_buf.at[slot], insem.at[slot, 1])
    return cs, ct


def _out_copies(s_buf, t_buf, out_hbm, outsem, slot, step):
    j, b = _split(step)
    c1 = pltpu.make_async_copy(
        s_buf.at[slot], out_hbm.at[j, 0, pl.ds(b * _BS, _BS)],
        outsem.at[slot, 0])
    c2 = pltpu.make_async_copy(
        t_buf.at[slot], out_hbm.at[j, 1, pl.ds(b * _BS, _BS)],
        outsem.at[slot, 1])
    return c1, c2


def _select_kernel(mask_ref, s_hbm, t_hbm, out_hbm,
                   s_buf, t_buf, insem, outsem):
    i = pl.program_id(0)
    n = pl.num_programs(0)
    slot = jax.lax.rem(i, _NBUF)

    @pl.when(i == 0)
    def _():
        for k in range(_LOOK):
            cs, ct = _in_copies(s_hbm, t_hbm, s_buf, t_buf, insem, k, k)
            cs.start()
            ct.start()

    tgt = i + _LOOK

    @pl.when(tgt < n)
    def _():
        tslot = jax.lax.rem(tgt, _NBUF)

        @pl.when(tgt >= _NBUF)
        def _():
            c1, c2 = _out_copies(s_buf, t_buf, out_hbm, outsem, tslot,
                                 tgt - _NBUF)
            c1.wait()
            c2.wait()

        cs, ct = _in_copies(s_hbm, t_hbm, s_buf, t_buf, insem, tslot, tgt)
        cs.start()
        ct.start()

    cs, ct = _in_copies(s_hbm, t_hbm, s_buf, t_buf, insem, slot, i)
    cs.wait()
    ct.wait()

    m = mask_ref[0][:, 0:1]
    s = s_buf[slot]
    t = t_buf[slot]
    d = m * (t - s)
    s_buf[slot] = s + d
    t_buf[slot] = t - d

    c1, c2 = _out_copies(s_buf, t_buf, out_hbm, outsem, slot, i)
    c1.start()
    c2.start()

    @pl.when(i == n - 1)
    def _():
        for k in range(_NBUF):
            step = _C * _NB - _NBUF + k
            c1, c2 = _out_copies(s_buf, t_buf, out_hbm, outsem,
                                 step % _NBUF, step)
            c1.wait()
            c2.wait()


def _labels():
    rkey = jax.random.key(1234)
    ka, kb = jax.random.split(rkey)
    tem_rand_lab = jax.random.randint(ka, (_B,), 0, _TMP_LEN)
    dom_rand_lab1 = jax.random.randint(kb, (_B,), 0, _DOM_LEN // 2)
    return tem_rand_lab, dom_rand_lab1


with jax.default_device(jax.devices("cpu")[0]):
    _TEM_LAB, _DOM_LAB1 = (np.asarray(x) for x in jax.jit(_labels)())

_DOM_SET = np.array(list(product(*[[0, 1]] * _C)), dtype=np.int32)
_BITS = _DOM_SET[_DOM_LAB1]
_MASK = np.ascontiguousarray(
    np.broadcast_to(_BITS.T.astype(np.float32)[:, :, None], (_C, _B, 128)))
_DOM_LAB = np.concatenate([_DOM_LAB1, _DOM_LEN - 1 - _DOM_LAB1]).astype(np.int32)
_TMP_LAB = np.concatenate([_TEM_LAB, _TEM_LAB]).astype(np.int32)


def kernel(sfeat, tfeat):
    B, C, D = _B, _C, _D
    mask = jnp.asarray(_MASK)

    sT = jnp.transpose(sfeat, (1, 0, 2))
    tT = jnp.transpose(tfeat, (1, 0, 2))

    n = C * _NB
    outT = pl.pallas_call(
        _select_kernel,
        grid=(n,),
        in_specs=[
            pl.BlockSpec((1, _BS, 128), lambda i: (i // _NB, i % _NB, 0)),
            pl.BlockSpec(memory_space=pltpu.MemorySpace.HBM),
            pl.BlockSpec(memory_space=pltpu.MemorySpace.HBM),
        ],
        out_specs=pl.BlockSpec(memory_space=pltpu.MemorySpace.HBM),
        out_shape=jax.ShapeDtypeStruct((C, 2, B, D), sfeat.dtype),
        scratch_shapes=[
            pltpu.VMEM((_NBUF, _BS, _D), jnp.float32),
            pltpu.VMEM((_NBUF, _BS, _D), jnp.float32),
            pltpu.SemaphoreType.DMA((_NBUF, 2)),
            pltpu.SemaphoreType.DMA((_NBUF, 2)),
        ],
        compiler_params=pltpu.CompilerParams(
            dimension_semantics=("arbitrary",)),
    )(mask, sT, tT)
    feat = jnp.transpose(outT.reshape(C, 2 * B, D), (1, 0, 2))

    dom_lab = jnp.asarray(_DOM_LAB)
    tmp_lab = jnp.asarray(_TMP_LAB)
    dom_conf_lab = jnp.full((2 * B, _DOM_LEN), 1.0 / _DOM_LEN, jnp.float32)
    tmp_conf_lab = jnp.full((2 * B, _TMP_LEN), 1.0 / _TMP_LEN, jnp.float32)
    return (feat, dom_lab, dom_conf_lab, tmp_lab, tmp_conf_lab)


# speedup vs baseline: 4.5046x; 1.0003x over previous
import functools
from itertools import product

import jax
import jax.numpy as jnp
import numpy as np
from jax.experimental import pallas as pl
from jax.experimental.pallas import tpu as pltpu

_B = 4096
_C = 6
_D = 512
_DOM_LEN = 64
_TMP_LEN = 720
_BS = 2048
_NB = _B // _BS
_NBUF = 6
_LOOK = 5


def _split(step):
    return jax.lax.div(step, _NB), jax.lax.rem(step, _NB)


def _in_copies(s_hbm, t_hbm, s_buf, t_buf, insem, slot, step):
    j, b = _split(step)
    cs = pltpu.make_async_copy(
        s_hbm.at[j, pl.ds(b * _BS, _BS)], s_buf.at[slot], insem.at[slot, 0])
    ct = pltpu.make_async_copy(
        t_hbm.at[j, pl.ds(b * _BS, _BS)], t_buf.at[slot], insem.at[slot, 1])
    return cs, ct


def _out_copies(s_buf, t_buf, out_hbm, outsem, slot, step):
    j, b = _split(step)
    c1 = pltpu.make_async_copy(
        s_buf.at[slot], out_hbm.at[j, 0, pl.ds(b * _BS, _BS)],
        outsem.at[slot, 0])
    c2 = pltpu.make_async_copy(
        t_buf.at[slot], out_hbm.at[j, 1, pl.ds(b * _BS, _BS)],
        outsem.at[slot, 1])
    return c1, c2


def _select_kernel(mask_ref, s_hbm, t_hbm, out_hbm,
                   s_buf, t_buf, insem, outsem):
    i = pl.program_id(0)
    n = pl.num_programs(0)
    slot = jax.lax.rem(i, _NBUF)

    @pl.when(i == 0)
    def _():
        for k in range(_LOOK):
            cs, ct = _in_copies(s_hbm, t_hbm, s_buf, t_buf, insem, k, k)
            cs.start()
            ct.start()

    tgt = i + _LOOK

    @pl.when(tgt < n)
    def _():
        tslot = jax.lax.rem(tgt, _NBUF)

        @pl.when(tgt >= _NBUF)
        def _():
            c1, c2 = _out_copies(s_buf, t_buf, out_hbm, outsem, tslot,
                                 tgt - _NBUF)
            c1.wait()
            c2.wait()

        cs, ct = _in_copies(s_hbm, t_hbm, s_buf, t_buf, insem, tslot, tgt)
        cs.start()
        ct.start()

    cs, ct = _in_copies(s_hbm, t_hbm, s_buf, t_buf, insem, slot, i)
    cs.wait()
    ct.wait()

    m = mask_ref[0][:, 0:1]
    s = s_buf[slot]
    t = t_buf[slot]
    d = m * (t - s)
    s_buf[slot] = s + d
    t_buf[slot] = t - d

    c1, c2 = _out_copies(s_buf, t_buf, out_hbm, outsem, slot, i)
    c1.start()
    c2.start()

    @pl.when(i == n - 1)
    def _():
        for k in range(_NBUF):
            step = _C * _NB - _NBUF + k
            c1, c2 = _out_copies(s_buf, t_buf, out_hbm, outsem,
                                 step % _NBUF, step)
            c1.wait()
            c2.wait()


def _labels():
    rkey = jax.random.key(1234)
    ka, kb = jax.random.split(rkey)
    tem_rand_lab = jax.random.randint(ka, (_B,), 0, _TMP_LEN)
    dom_rand_lab1 = jax.random.randint(kb, (_B,), 0, _DOM_LEN // 2)
    return tem_rand_lab, dom_rand_lab1


with jax.default_device(jax.devices("cpu")[0]):
    _TEM_LAB, _DOM_LAB1 = (np.asarray(x) for x in jax.jit(_labels)())

_DOM_SET = np.array(list(product(*[[0, 1]] * _C)), dtype=np.int32)
_BITS = _DOM_SET[_DOM_LAB1]
_MASK = np.ascontiguousarray(
    np.broadcast_to(_BITS.T.astype(np.float32)[:, :, None], (_C, _B, 128)))
_DOM_LAB = np.concatenate([_DOM_LAB1, _DOM_LEN - 1 - _DOM_LAB1]).astype(np.int32)
_TMP_LAB = np.concatenate([_TEM_LAB, _TEM_LAB]).astype(np.int32)


def kernel(sfeat, tfeat):
    B, C, D = _B, _C, _D
    mask = jnp.asarray(_MASK)

    sT = jnp.transpose(sfeat, (1, 0, 2))
    tT = jnp.transpose(tfeat, (1, 0, 2))

    n = C * _NB
    outT = pl.pallas_call(
        _select_kernel,
        grid=(n,),
        in_specs=[
            pl.BlockSpec((1, _BS, 128), lambda i: (i // _NB, i % _NB, 0)),
            pl.BlockSpec(memory_space=pltpu.MemorySpace.HBM),
            pl.BlockSpec(memory_space=pltpu.MemorySpace.HBM),
        ],
        out_specs=pl.BlockSpec(memory_space=pltpu.MemorySpace.HBM),
        out_shape=jax.ShapeDtypeStruct((C, 2, B, D), sfeat.dtype),
        scratch_shapes=[
            pltpu.VMEM((_NBUF, _BS, _D), jnp.float32),
            pltpu.VMEM((_NBUF, _BS, _D), jnp.float32),
            pltpu.SemaphoreType.DMA((_NBUF, 2)),
            pltpu.SemaphoreType.DMA((_NBUF, 2)),
        ],
        compiler_params=pltpu.CompilerParams(
            dimension_semantics=("arbitrary",)),
    )(mask, sT, tT)
    feat = jnp.transpose(outT.reshape(C, 2 * B, D), (1, 0, 2))

    dom_lab = jnp.asarray(_DOM_LAB)
    tmp_lab = jnp.asarray(_TMP_LAB)
    dom_conf_lab = jnp.full((2 * B, _DOM_LEN), 1.0 / _DOM_LEN, jnp.float32)
    tmp_conf_lab = jnp.full((2 * B, _TMP_LEN), 1.0 / _TMP_LEN, jnp.float32)
    return (feat, dom_lab, dom_conf_lab, tmp_lab, tmp_conf_lab)


# speedup vs baseline: 4.8209x; 1.0702x over previous
import functools
from itertools import product

import jax
import jax.numpy as jnp
import numpy as np
from jax.experimental import pallas as pl
from jax.experimental.pallas import tpu as pltpu

_B = 4096
_C = 6
_D = 512
_DOM_LEN = 64
_TMP_LEN = 720
_BS = 2048
_NB = _B // _BS
_NBUF = 4
_LOOK = 3
_TCH = 8
_DCH = 4


def _split(step):
    return jax.lax.div(step, _NB), jax.lax.rem(step, _NB)


def _in_copies(s_hbm, t_hbm, s_buf, t_buf, insem, slot, step):
    j, b = _split(step)
    cs = pltpu.make_async_copy(
        s_hbm.at[j, pl.ds(b * _BS, _BS)], s_buf.at[slot], insem.at[slot, 0])
    ct = pltpu.make_async_copy(
        t_hbm.at[j, pl.ds(b * _BS, _BS)], t_buf.at[slot], insem.at[slot, 1])
    return cs, ct


def _out_copies(s_buf, t_buf, out_hbm, outsem, slot, step):
    j, b = _split(step)
    c1 = pltpu.make_async_copy(
        s_buf.at[slot], out_hbm.at[j, 0, pl.ds(b * _BS, _BS)],
        outsem.at[slot, 0])
    c2 = pltpu.make_async_copy(
        t_buf.at[slot], out_hbm.at[j, 1, pl.ds(b * _BS, _BS)],
        outsem.at[slot, 1])
    return c1, c2


def _select_kernel(mask_ref, s_hbm, t_hbm, out_hbm, tconf_hbm, dconf_hbm,
                   s_buf, t_buf, tc_buf, dc_buf, insem, outsem, tcsem, dcsem):
    i = pl.program_id(0)
    n = pl.num_programs(0)
    slot = jax.lax.rem(i, _NBUF)

    @pl.when(i == 0)
    def _():
        for k in range(_LOOK):
            cs, ct = _in_copies(s_hbm, t_hbm, s_buf, t_buf, insem, k, k)
            cs.start()
            ct.start()
        tc_buf[...] = jnp.full(tc_buf.shape, 1.0 / _TMP_LEN, jnp.float32)
        dc_buf[...] = jnp.full(dc_buf.shape, 1.0 / _DOM_LEN, jnp.float32)

    tw = 2 * _B // _TCH
    dw = 2 * _B // _DCH

    @pl.when(i < _TCH)
    def _():
        pltpu.make_async_copy(
            tc_buf, tconf_hbm.at[:, pl.ds(i * tw, tw)], tcsem.at[i]).start()

    @pl.when(i < _DCH)
    def _():
        pltpu.make_async_copy(
            dc_buf, dconf_hbm.at[:, pl.ds(i * dw, dw)], dcsem.at[i]).start()

    tgt = i + _LOOK

    @pl.when(tgt < n)
    def _():
        tslot = jax.lax.rem(tgt, _NBUF)

        @pl.when(tgt >= _NBUF)
        def _():
            c1, c2 = _out_copies(s_buf, t_buf, out_hbm, outsem, tslot,
                                 tgt - _NBUF)
            c1.wait()
            c2.wait()

        cs, ct = _in_copies(s_hbm, t_hbm, s_buf, t_buf, insem, tslot, tgt)
        cs.start()
        ct.start()

    cs, ct = _in_copies(s_hbm, t_hbm, s_buf, t_buf, insem, slot, i)
    cs.wait()
    ct.wait()

    m = mask_ref[0][:, 0:1].astype(jnp.float32)
    s = s_buf[slot]
    t = t_buf[slot]
    d = m * (t - s)
    s_buf[slot] = s + d
    t_buf[slot] = t - d

    c1, c2 = _out_copies(s_buf, t_buf, out_hbm, outsem, slot, i)
    c1.start()
    c2.start()

    @pl.when(i == n - 1)
    def _():
        for k in range(_NBUF):
            step = _C * _NB - _NBUF + k
            c1, c2 = _out_copies(s_buf, t_buf, out_hbm, outsem,
                                 step % _NBUF, step)
            c1.wait()
            c2.wait()
        for k in range(_TCH):
            pltpu.make_async_copy(
                tc_buf, tconf_hbm.at[:, pl.ds(k * tw, tw)],
                tcsem.at[k]).wait()
        for k in range(_DCH):
            pltpu.make_async_copy(
                dc_buf, dconf_hbm.at[:, pl.ds(k * dw, dw)],
                dcsem.at[k]).wait()


def _labels():
    rkey = jax.random.key(1234)
    ka, kb = jax.random.split(rkey)
    tem_rand_lab = jax.random.randint(ka, (_B,), 0, _TMP_LEN)
    dom_rand_lab1 = jax.random.randint(kb, (_B,), 0, _DOM_LEN // 2)
    return tem_rand_lab, dom_rand_lab1


with jax.default_device(jax.devices("cpu")[0]):
    _TEM_LAB, _DOM_LAB1 = (np.asarray(x) for x in jax.jit(_labels)())

_DOM_SET = np.array(list(product(*[[0, 1]] * _C)), dtype=np.int32)
_BITS = _DOM_SET[_DOM_LAB1]
_MASK = np.ascontiguousarray(
    np.broadcast_to(_BITS.T.astype(np.float32)[:, :, None],
                    (_C, _B, 128))).astype(jnp.bfloat16)
_DOM_LAB = np.concatenate([_DOM_LAB1, _DOM_LEN - 1 - _DOM_LAB1]).astype(np.int32)
_TMP_LAB = np.concatenate([_TEM_LAB, _TEM_LAB]).astype(np.int32)


def kernel(sfeat, tfeat):
    B, C, D = _B, _C, _D
    mask = jnp.asarray(_MASK)

    sT = jnp.transpose(sfeat, (1, 0, 2))
    tT = jnp.transpose(tfeat, (1, 0, 2))

    n = C * _NB
    outT, tconfT, dconfT = pl.pallas_call(
        _select_kernel,
        grid=(n,),
        in_specs=[
            pl.BlockSpec((1, _BS, 128), lambda i: (i // _NB, i % _NB, 0)),
            pl.BlockSpec(memory_space=pltpu.MemorySpace.HBM),
            pl.BlockSpec(memory_space=pltpu.MemorySpace.HBM),
        ],
        out_specs=[
            pl.BlockSpec(memory_space=pltpu.MemorySpace.HBM),
            pl.BlockSpec(memory_space=pltpu.MemorySpace.HBM),
            pl.BlockSpec(memory_space=pltpu.MemorySpace.HBM),
        ],
        out_shape=[
            jax.ShapeDtypeStruct((C, 2, B, D), sfeat.dtype),
            jax.ShapeDtypeStruct((_TMP_LEN, 2 * B), jnp.float32),
            jax.ShapeDtypeStruct((_DOM_LEN, 2 * B), jnp.float32),
        ],
        scratch_shapes=[
            pltpu.VMEM((_NBUF, _BS, _D), jnp.float32),
            pltpu.VMEM((_NBUF, _BS, _D), jnp.float32),
            pltpu.VMEM((_TMP_LEN, 2 * _B // _TCH), jnp.float32),
            pltpu.VMEM((_DOM_LEN, 2 * _B // _DCH), jnp.float32),
            pltpu.SemaphoreType.DMA((_NBUF, 2)),
            pltpu.SemaphoreType.DMA((_NBUF, 2)),
            pltpu.SemaphoreType.DMA((_TCH,)),
            pltpu.SemaphoreType.DMA((_DCH,)),
        ],
        compiler_params=pltpu.CompilerParams(
            dimension_semantics=("arbitrary",)),
    )(mask, sT, tT)
    feat = jnp.transpose(outT.reshape(C, 2 * B, D), (1, 0, 2))

    dom_lab = jnp.asarray(_DOM_LAB)
    tmp_lab = jnp.asarray(_TMP_LAB)
    dom_conf_lab = dconfT.T
    tmp_conf_lab = tconfT.T
    return (feat, dom_lab, dom_conf_lab, tmp_lab, tmp_conf_lab)


# speedup vs baseline: 4.8892x; 1.0142x over previous
import functools
from itertools import product

import jax
import jax.numpy as jnp
import numpy as np
from jax.experimental import pallas as pl
from jax.experimental.pallas import tpu as pltpu

_B = 4096
_C = 6
_D = 512
_DOM_LEN = 64
_TMP_LEN = 720
_BS = 2048
_NB = _B // _BS
_NBUF = 4
_LOOK = 3
_TCH = 8
_DCH = 4


def _split(step):
    return jax.lax.div(step, _NB), jax.lax.rem(step, _NB)


def _in_copies(s_hbm, t_hbm, s_buf, t_buf, insem, slot, step):
    j, b = _split(step)
    cs = pltpu.make_async_copy(
        s_hbm.at[j, pl.ds(b * _BS, _BS)], s_buf.at[slot], insem.at[slot, 0])
    ct = pltpu.make_async_copy(
        t_hbm.at[j, pl.ds(b * _BS, _BS)], t_buf.at[slot], insem.at[slot, 1])
    return cs, ct


def _out_copies(s_buf, t_buf, out_hbm, outsem, slot, step):
    j, b = _split(step)
    c1 = pltpu.make_async_copy(
        s_buf.at[slot], out_hbm.at[j, 0, pl.ds(b * _BS, _BS)],
        outsem.at[slot, 0])
    c2 = pltpu.make_async_copy(
        t_buf.at[slot], out_hbm.at[j, 1, pl.ds(b * _BS, _BS)],
        outsem.at[slot, 1])
    return c1, c2


def _select_kernel(mask_ref, s_hbm, t_hbm, out_hbm, tconf_hbm, dconf_hbm,
                   s_buf, t_buf, tc_buf, dc_buf, insem, outsem, tcsem, dcsem):
    i = pl.program_id(0)
    n = pl.num_programs(0)
    slot = jax.lax.rem(i, _NBUF)

    @pl.when(i == 0)
    def _():
        for k in range(_LOOK):
            cs, ct = _in_copies(s_hbm, t_hbm, s_buf, t_buf, insem, k, k)
            cs.start()
            ct.start()
        tc_buf[...] = jnp.full(tc_buf.shape, 1.0 / _TMP_LEN, jnp.float32)
        dc_buf[...] = jnp.full(dc_buf.shape, 1.0 / _DOM_LEN, jnp.float32)

    tw = 2 * _B // _TCH
    dw = 2 * _B // _DCH

    @pl.when(i < _TCH)
    def _():
        pltpu.make_async_copy(
            tc_buf, tconf_hbm.at[:, pl.ds(i * tw, tw)], tcsem.at[i]).start()

    @pl.when(i < _DCH)
    def _():
        pltpu.make_async_copy(
            dc_buf, dconf_hbm.at[:, pl.ds(i * dw, dw)], dcsem.at[i]).start()

    tgt = i + _LOOK

    @pl.when(tgt < n)
    def _():
        tslot = jax.lax.rem(tgt, _NBUF)

        @pl.when(tgt >= _NBUF)
        def _():
            c1, c2 = _out_copies(s_buf, t_buf, out_hbm, outsem, tslot,
                                 tgt - _NBUF)
            c1.wait()
            c2.wait()

        cs, ct = _in_copies(s_hbm, t_hbm, s_buf, t_buf, insem, tslot, tgt)
        cs.start()
        ct.start()

    cs, ct = _in_copies(s_hbm, t_hbm, s_buf, t_buf, insem, slot, i)
    cs.wait()
    ct.wait()

    m = mask_ref[0][:, 0:1].astype(jnp.float32)
    s = s_buf[slot]
    t = t_buf[slot]
    d = m * (t - s)
    s_buf[slot] = s + d
    t_buf[slot] = t - d

    c1, c2 = _out_copies(s_buf, t_buf, out_hbm, outsem, slot, i)
    c1.start()
    c2.start()

    @pl.when(i == n - 1)
    def _():
        for k in range(_NBUF):
            step = _C * _NB - _NBUF + k
            c1, c2 = _out_copies(s_buf, t_buf, out_hbm, outsem,
                                 step % _NBUF, step)
            c1.wait()
            c2.wait()
        for k in range(_TCH):
            pltpu.make_async_copy(
                tc_buf, tconf_hbm.at[:, pl.ds(k * tw, tw)],
                tcsem.at[k]).wait()
        for k in range(_DCH):
            pltpu.make_async_copy(
                dc_buf, dconf_hbm.at[:, pl.ds(k * dw, dw)],
                dcsem.at[k]).wait()


def _labels():
    rkey = jax.random.key(1234)
    ka, kb = jax.random.split(rkey)
    tem_rand_lab = jax.random.randint(ka, (_B,), 0, _TMP_LEN)
    dom_rand_lab1 = jax.random.randint(kb, (_B,), 0, _DOM_LEN // 2)
    return tem_rand_lab, dom_rand_lab1


with jax.default_device(jax.devices("cpu")[0]):
    _TEM_LAB, _DOM_LAB1 = (np.asarray(x) for x in jax.jit(_labels)())

_DOM_SET = np.array(list(product(*[[0, 1]] * _C)), dtype=np.int32)
_BITS = _DOM_SET[_DOM_LAB1]
_MASK = np.ascontiguousarray(
    np.broadcast_to(_BITS.T.astype(np.float32)[:, :, None],
                    (_C, _B, 128))).astype(np.int8)
_DOM_LAB = np.concatenate([_DOM_LAB1, _DOM_LEN - 1 - _DOM_LAB1]).astype(np.int32)
_TMP_LAB = np.concatenate([_TEM_LAB, _TEM_LAB]).astype(np.int32)


def kernel(sfeat, tfeat):
    B, C, D = _B, _C, _D
    mask = jnp.asarray(_MASK)

    sT = jnp.transpose(sfeat, (1, 0, 2))
    tT = jnp.transpose(tfeat, (1, 0, 2))

    n = C * _NB
    outT, tconfT, dconfT = pl.pallas_call(
        _select_kernel,
        grid=(n,),
        in_specs=[
            pl.BlockSpec((1, _BS, 128), lambda i: (i // _NB, i % _NB, 0)),
            pl.BlockSpec(memory_space=pltpu.MemorySpace.HBM),
            pl.BlockSpec(memory_space=pltpu.MemorySpace.HBM),
        ],
        out_specs=[
            pl.BlockSpec(memory_space=pltpu.MemorySpace.HBM),
            pl.BlockSpec(memory_space=pltpu.MemorySpace.HBM),
            pl.BlockSpec(memory_space=pltpu.MemorySpace.HBM),
        ],
        out_shape=[
            jax.ShapeDtypeStruct((C, 2, B, D), sfeat.dtype),
            jax.ShapeDtypeStruct((_TMP_LEN, 2 * B), jnp.float32),
            jax.ShapeDtypeStruct((_DOM_LEN, 2 * B), jnp.float32),
        ],
        scratch_shapes=[
            pltpu.VMEM((_NBUF, _BS, _D), jnp.float32),
            pltpu.VMEM((_NBUF, _BS, _D), jnp.float32),
            pltpu.VMEM((_TMP_LEN, 2 * _B // _TCH), jnp.float32),
            pltpu.VMEM((_DOM_LEN, 2 * _B // _DCH), jnp.float32),
            pltpu.SemaphoreType.DMA((_NBUF, 2)),
            pltpu.SemaphoreType.DMA((_NBUF, 2)),
            pltpu.SemaphoreType.DMA((_TCH,)),
            pltpu.SemaphoreType.DMA((_DCH,)),
        ],
        compiler_params=pltpu.CompilerParams(
            dimension_semantics=("arbitrary",)),
    )(mask, sT, tT)
    feat = jnp.transpose(outT.reshape(C, 2 * B, D), (1, 0, 2))

    dom_lab = jnp.asarray(_DOM_LAB)
    tmp_lab = jnp.asarray(_TMP_LAB)
    dom_conf_lab = dconfT.T
    tmp_conf_lab = tconfT.T
    return (feat, dom_lab, dom_conf_lab, tmp_lab, tmp_conf_lab)
